# baseline (speedup 1.0000x reference)
import functools

import jax
import jax.numpy as jnp
from jax import lax
from jax.experimental import pallas as pl
from jax.experimental.pallas import tpu as pltpu
from jax.experimental.pallas import tpu_sc as plsc

_LANES = 16
_IDXCAP = 128
_NBUF = 8


_WB = 8192


def _project_table_tc(emb_table, fc_w):
    V, D = emb_table.shape
    C = fc_w.shape[0]
    nb = pl.cdiv(V, _WB)
    PR = _WB // 8

    tT = emb_table.T

    def body(tT_ref, w_ref, out_ref):
        xT = tT_ref[...].astype(jnp.bfloat16).T
        wT = w_ref[...].astype(jnp.bfloat16).T
        for k in range(8):
            tk = lax.dot_general(xT[k * PR:(k + 1) * PR, :], wT,
                                 (((1,), (0,)), ((), ())),
                                 preferred_element_type=jnp.float32)
            out_ref[:, pl.ds(k * C, C)] = tk

    out = pl.pallas_call(
        body,
        grid=(nb,),
        in_specs=[
            pl.BlockSpec((D, _WB), lambda i: (0, i)),
            pl.BlockSpec((C, D), lambda i: (0, 0)),
        ],
        out_specs=pl.BlockSpec((PR, 8 * C), lambda i: (i, 0)),
        out_shape=jax.ShapeDtypeStruct((nb * PR, 8 * C), jnp.float32),
        compiler_params=pltpu.CompilerParams(fuse_transposed_lhs_in_matmul=True),
    )(tT, fc_w)
    return out.reshape(nb * _WB, C)


def _gather_sum_sc(input_ids, tprime):
    B, L = input_ids.shape
    _, C = tprime.shape
    info = plsc.get_sparse_core_info()
    nc, ns = info.num_cores, info.num_subcores
    NW = nc * ns
    IPW = B // NW
    GB = 32

    ids_flat = input_ids.reshape(B * L)
    mesh = plsc.VectorSubcoreMesh(core_axis_name="c", subcore_axis_name="s")

    LP = ((L + _LANES - 1) // _LANES) * _LANES

    @functools.partial(
        pl.kernel,
        out_type=jax.ShapeDtypeStruct((C, B), jnp.float32),
        mesh=mesh,
        scratch_types=(
            [pltpu.VMEM((LP,), jnp.int32) for _ in range(_NBUF)]
            + [
                pltpu.VMEM((_NBUF, L, C), jnp.float32),
                pltpu.VMEM((C, GB), jnp.float32),
            ]
            + [pltpu.SemaphoreType.DMA for _ in range(2 * _NBUF)]
        ),
        compiler_params=pltpu.CompilerParams(use_tc_tiling_on_sc=False,
                                             needs_layout_passes=False),
    )
    def k(ids_hbm, tp_hbm, out_hbm, *bufs):
        idxs = bufs[:_NBUF]
        rows_v, stage_v = bufs[_NBUF], bufs[_NBUF + 1]
        gsems = bufs[_NBUF + 2:2 * _NBUF + 2]
        isems = bufs[2 * _NBUF + 2:]
        wid = lax.axis_index("s") * nc + lax.axis_index("c")
        base = wid * IPW
        lane_iota = lax.iota(jnp.int32, _LANES)
        zeros16 = jnp.zeros((_LANES,), jnp.int32)

        def idx_copy(it, p):
            return pltpu.make_async_copy(
                ids_hbm.at[pl.ds(it * L, L)], idxs[p].at[pl.ds(0, L)],
                isems[p])

        PRS = (_WB // 8).bit_length() - 1

        def remap_ids(p):
            for q in range(LP // _LANES):
                v = idxs[p][pl.ds(q * _LANES, _LANES)]
                r = v & (_WB - 1)
                rho = (v & ~(_WB - 1)) + ((r & (_WB // 8 - 1)) << 3) + (r >> PRS)
                idxs[p][pl.ds(q * _LANES, _LANES)] = rho

        def gather_parts(p):
            parts = []
            for off in range(0, L, _IDXCAP):
                n = min(_IDXCAP, L - off)
                parts.append((idxs[p].at[pl.ds(off, n)],
                              rows_v.at[p, pl.ds(off, n)]))
            return parts

        def start_gathers(p):
            for idx_s, dst_s in gather_parts(p):
                pltpu.async_copy(tp_hbm.at[idx_s], dst_s, gsems[p])

        def wait_gathers(p):
            for idx_s, dst_s in gather_parts(p):
                pltpu.make_async_copy(tp_hbm.at[idx_s], dst_s, gsems[p]).wait()

        for u in range(_NBUF - 1):
            idx_copy(base + u, u).start()
            idx_copy(base + u, u).wait()
            remap_ids(u)
            start_gathers(u)
        idx_copy(base + _NBUF - 1, _NBUF - 1).start()

        def quad_body(i4, carry):
            for p in range(_NBUF):
                it_off = i4 * _NBUF + p
                it = base + it_off

                @pl.when(it_off + _NBUF - 1 < IPW)
                def _():
                    idx_copy(it + _NBUF - 1, (p + _NBUF - 1) % _NBUF).wait()
                    remap_ids((p + _NBUF - 1) % _NBUF)
                    start_gathers((p + _NBUF - 1) % _NBUF)

                wait_gathers(p)

                @pl.when(it_off + _NBUF < IPW)
                def _():
                    idx_copy(it + _NBUF, p).start()

                def acc_body(r, acc):
                    return acc + rows_v[p, r, pl.ds(0, _LANES)]

                z = jnp.zeros((_LANES,), jnp.float32)
                acc = lax.fori_loop(0, L, acc_body, z, unroll=8)

                g = lax.rem(it_off, GB)
                plsc.store_scatter(stage_v, [lane_iota, zeros16 + g], acc)

                @pl.when(lax.rem(it_off + 1, GB) == 0)
                def _():
                    dst = pl.multiple_of(it + 1 - GB, GB)
                    pltpu.sync_copy(stage_v, out_hbm.at[:, pl.ds(dst, GB)])
            return carry

        lax.fori_loop(0, IPW // _NBUF, quad_body, 0)

    return k(ids_flat, tprime)


def _loss_tc(sums_cm, labels, fc_b, L):
    C, B = sums_cm.shape
    BB = 4096
    nb = B // BB
    inv = float(1.0 / L)

    def body(sum_ref, lab_ref, b_ref, logits_ref, loss_ref):
        i = pl.program_id(0)
        logits = sum_ref[...] * inv + b_ref[...]
        logits_ref[...] = logits
        m = jnp.max(logits, axis=0, keepdims=True)
        lse = jnp.log(jnp.sum(jnp.exp(logits - m), axis=0, keepdims=True)) + m
        onehot = lab_ref[...] == lax.broadcasted_iota(jnp.int32, logits.shape, 0)
        ll = jnp.sum(jnp.where(onehot, logits, 0.0), axis=0, keepdims=True)
        part = jnp.sum(lse - ll)

        @pl.when(i == 0)
        def _():
            loss_ref[0, 0] = 0.0

        loss_ref[0, 0] += part

        @pl.when(i == nb - 1)
        def _():
            loss_ref[0, 0] = loss_ref[0, 0] / B

    logits_cm, loss = pl.pallas_call(
        body,
        grid=(nb,),
        in_specs=[
            pl.BlockSpec((C, BB), lambda i: (0, i)),
            pl.BlockSpec((1, BB), lambda i: (0, i)),
            pl.BlockSpec((C, 1), lambda i: (0, 0)),
        ],
        out_specs=[
            pl.BlockSpec((C, BB), lambda i: (0, i)),
            pl.BlockSpec(memory_space=pltpu.SMEM),
        ],
        out_shape=[
            jax.ShapeDtypeStruct((C, B), jnp.float32),
            jax.ShapeDtypeStruct((1, 1), jnp.float32),
        ],
    )(sums_cm, labels.reshape(1, B), fc_b.reshape(C, 1))
    return loss[0, 0], logits_cm.T


def kernel(input_ids, labels, emb_table, fc_w, fc_b):
    L = input_ids.shape[1]
    tprime = _project_table_tc(emb_table, fc_w)
    sums = _gather_sum_sc(input_ids, tprime)
    loss, logits = _loss_tc(sums, labels, fc_b, L)
    return loss, logits


# speedup vs baseline: 1.0450x; 1.0450x over previous
import functools

import jax
import jax.numpy as jnp
from jax import lax
from jax.experimental import pallas as pl
from jax.experimental.pallas import tpu as pltpu
from jax.experimental.pallas import tpu_sc as plsc

_LANES = 16
_IDXCAP = 128
_NBUF = 4


_WB = 16384


def _project_table_tc(emb_table, fc_w):
    V, D = emb_table.shape
    C = fc_w.shape[0]
    nb = pl.cdiv(V, _WB)
    PR = _WB // 8

    tT = emb_table.T

    def body(tT_ref, w_ref, out_ref):
        xT = tT_ref[...].astype(jnp.bfloat16).T
        wT = w_ref[...].astype(jnp.bfloat16).T
        for k in range(8):
            tk = lax.dot_general(xT[k * PR:(k + 1) * PR, :], wT,
                                 (((1,), (0,)), ((), ())),
                                 preferred_element_type=jnp.float32)
            out_ref[:, pl.ds(k * C, C)] = tk

    out = pl.pallas_call(
        body,
        grid=(nb,),
        in_specs=[
            pl.BlockSpec((D, _WB), lambda i: (0, i)),
            pl.BlockSpec((C, D), lambda i: (0, 0)),
        ],
        out_specs=pl.BlockSpec((PR, 8 * C), lambda i: (i, 0)),
        out_shape=jax.ShapeDtypeStruct((nb * PR, 8 * C), jnp.float32),
        compiler_params=pltpu.CompilerParams(fuse_transposed_lhs_in_matmul=True),
    )(tT, fc_w)
    return out.reshape(nb * _WB, C)


def _gather_sum_sc(input_ids, tprime):
    B, L = input_ids.shape
    _, C = tprime.shape
    info = plsc.get_sparse_core_info()
    nc, ns = info.num_cores, info.num_subcores
    NW = nc * ns
    IPW = B // NW
    GB = 32

    ids_flat = input_ids.reshape(B * L)
    mesh = plsc.VectorSubcoreMesh(core_axis_name="c", subcore_axis_name="s")

    LP = ((L + _LANES - 1) // _LANES) * _LANES

    @functools.partial(
        pl.kernel,
        out_type=jax.ShapeDtypeStruct((C, B), jnp.float32),
        mesh=mesh,
        scratch_types=(
            [pltpu.VMEM((LP,), jnp.int32) for _ in range(_NBUF)]
            + [
                pltpu.VMEM((_NBUF, L, C), jnp.float32),
                pltpu.VMEM((C, GB), jnp.float32),
            ]
            + [pltpu.SemaphoreType.DMA for _ in range(2 * _NBUF)]
        ),
        compiler_params=pltpu.CompilerParams(use_tc_tiling_on_sc=False,
                                             needs_layout_passes=False),
    )
    def k(ids_hbm, tp_hbm, out_hbm, *bufs):
        idxs = bufs[:_NBUF]
        rows_v, stage_v = bufs[_NBUF], bufs[_NBUF + 1]
        gsems = bufs[_NBUF + 2:2 * _NBUF + 2]
        isems = bufs[2 * _NBUF + 2:]
        wid = lax.axis_index("s") * nc + lax.axis_index("c")
        base = wid * IPW
        lane_iota = lax.iota(jnp.int32, _LANES)
        zeros16 = jnp.zeros((_LANES,), jnp.int32)

        def idx_copy(it, p):
            return pltpu.make_async_copy(
                ids_hbm.at[pl.ds(it * L, L)], idxs[p].at[pl.ds(0, L)],
                isems[p])

        PRS = (_WB // 8).bit_length() - 1

        def remap_ids(p):
            for q in range(LP // _LANES):
                v = idxs[p][pl.ds(q * _LANES, _LANES)]
                r = v & (_WB - 1)
                rho = (v & ~(_WB - 1)) + ((r & (_WB // 8 - 1)) << 3) + (r >> PRS)
                idxs[p][pl.ds(q * _LANES, _LANES)] = rho

        def gather_parts(p):
            parts = []
            for off in range(0, L, _IDXCAP):
                n = min(_IDXCAP, L - off)
                parts.append((idxs[p].at[pl.ds(off, n)],
                              rows_v.at[p, pl.ds(off, n)]))
            return parts

        def start_gathers(p):
            for idx_s, dst_s in gather_parts(p):
                pltpu.async_copy(tp_hbm.at[idx_s], dst_s, gsems[p])

        def wait_gathers(p):
            for idx_s, dst_s in gather_parts(p):
                pltpu.make_async_copy(tp_hbm.at[idx_s], dst_s, gsems[p]).wait()

        for u in range(_NBUF - 1):
            idx_copy(base + u, u).start()
            idx_copy(base + u, u).wait()
            remap_ids(u)
            start_gathers(u)
        idx_copy(base + _NBUF - 1, _NBUF - 1).start()

        def quad_body(i4, carry):
            for p in range(_NBUF):
                it_off = i4 * _NBUF + p
                it = base + it_off

                @pl.when(it_off + _NBUF - 1 < IPW)
                def _():
                    idx_copy(it + _NBUF - 1, (p + _NBUF - 1) % _NBUF).wait()
                    remap_ids((p + _NBUF - 1) % _NBUF)
                    start_gathers((p + _NBUF - 1) % _NBUF)

                wait_gathers(p)

                @pl.when(it_off + _NBUF < IPW)
                def _():
                    idx_copy(it + _NBUF, p).start()

                def acc_body(r, acc):
                    return acc + rows_v[p, r, pl.ds(0, _LANES)]

                z = jnp.zeros((_LANES,), jnp.float32)
                acc = lax.fori_loop(0, L, acc_body, z, unroll=8)

                g = lax.rem(it_off, GB)
                plsc.store_scatter(stage_v, [lane_iota, zeros16 + g], acc)

                @pl.when(lax.rem(it_off + 1, GB) == 0)
                def _():
                    dst = pl.multiple_of(it + 1 - GB, GB)
                    pltpu.sync_copy(stage_v, out_hbm.at[:, pl.ds(dst, GB)])
            return carry

        lax.fori_loop(0, IPW // _NBUF, quad_body, 0)

    return k(ids_flat, tprime)


def _loss_tc(sums_cm, labels, fc_b, L):
    C, B = sums_cm.shape
    BB = 4096
    nb = B // BB
    inv = float(1.0 / L)

    def body(sum_ref, lab_ref, b_ref, logits_ref, loss_ref):
        i = pl.program_id(0)
        logits = sum_ref[...] * inv + b_ref[...]
        logits_ref[...] = logits
        m = jnp.max(logits, axis=0, keepdims=True)
        lse = jnp.log(jnp.sum(jnp.exp(logits - m), axis=0, keepdims=True)) + m
        onehot = lab_ref[...] == lax.broadcasted_iota(jnp.int32, logits.shape, 0)
        ll = jnp.sum(jnp.where(onehot, logits, 0.0), axis=0, keepdims=True)
        part = jnp.sum(lse - ll)

        @pl.when(i == 0)
        def _():
            loss_ref[0, 0] = 0.0

        loss_ref[0, 0] += part

        @pl.when(i == nb - 1)
        def _():
            loss_ref[0, 0] = loss_ref[0, 0] / B

    logits_cm, loss = pl.pallas_call(
        body,
        grid=(nb,),
        in_specs=[
            pl.BlockSpec((C, BB), lambda i: (0, i)),
            pl.BlockSpec((1, BB), lambda i: (0, i)),
            pl.BlockSpec((C, 1), lambda i: (0, 0)),
        ],
        out_specs=[
            pl.BlockSpec((C, BB), lambda i: (0, i)),
            pl.BlockSpec(memory_space=pltpu.SMEM),
        ],
        out_shape=[
            jax.ShapeDtypeStruct((C, B), jnp.float32),
            jax.ShapeDtypeStruct((1, 1), jnp.float32),
        ],
    )(sums_cm, labels.reshape(1, B), fc_b.reshape(C, 1))
    return loss[0, 0], logits_cm.T


def kernel(input_ids, labels, emb_table, fc_w, fc_b):
    L = input_ids.shape[1]
    tprime = _project_table_tc(emb_table, fc_w)
    sums = _gather_sum_sc(input_ids, tprime)
    loss, logits = _loss_tc(sums, labels, fc_b, L)
    return loss, logits


# speedup vs baseline: 1.0503x; 1.0051x over previous
import functools

import jax
import jax.numpy as jnp
from jax import lax
from jax.experimental import pallas as pl
from jax.experimental.pallas import tpu as pltpu
from jax.experimental.pallas import tpu_sc as plsc

_LANES = 16
_IDXCAP = 128
_NBUF = 4


_WB = 32768


def _project_table_tc(emb_table, fc_w):
    V, D = emb_table.shape
    C = fc_w.shape[0]
    nb = pl.cdiv(V, _WB)
    PR = _WB // 8

    tT = emb_table.T

    def body(tT_ref, w_ref, out_ref):
        xT = tT_ref[...].astype(jnp.bfloat16).T
        wT = w_ref[...].astype(jnp.bfloat16).T
        for k in range(8):
            tk = lax.dot_general(xT[k * PR:(k + 1) * PR, :], wT,
                                 (((1,), (0,)), ((), ())),
                                 preferred_element_type=jnp.float32)
            out_ref[:, pl.ds(k * C, C)] = tk

    out = pl.pallas_call(
        body,
        grid=(nb,),
        in_specs=[
            pl.BlockSpec((D, _WB), lambda i: (0, i)),
            pl.BlockSpec((C, D), lambda i: (0, 0)),
        ],
        out_specs=pl.BlockSpec((PR, 8 * C), lambda i: (i, 0)),
        out_shape=jax.ShapeDtypeStruct((nb * PR, 8 * C), jnp.float32),
        compiler_params=pltpu.CompilerParams(fuse_transposed_lhs_in_matmul=True),
    )(tT, fc_w)
    return out.reshape(nb * _WB, C)


def _gather_sum_sc(input_ids, tprime):
    B, L = input_ids.shape
    _, C = tprime.shape
    info = plsc.get_sparse_core_info()
    nc, ns = info.num_cores, info.num_subcores
    NW = nc * ns
    IPW = B // NW
    GB = 32

    ids_flat = input_ids.reshape(B * L)
    mesh = plsc.VectorSubcoreMesh(core_axis_name="c", subcore_axis_name="s")

    LP = ((L + _LANES - 1) // _LANES) * _LANES

    @functools.partial(
        pl.kernel,
        out_type=jax.ShapeDtypeStruct((C, B), jnp.float32),
        mesh=mesh,
        scratch_types=(
            [pltpu.VMEM((LP,), jnp.int32) for _ in range(_NBUF)]
            + [
                pltpu.VMEM((_NBUF, L, C), jnp.float32),
                pltpu.VMEM((C, GB), jnp.float32),
            ]
            + [pltpu.SemaphoreType.DMA for _ in range(2 * _NBUF)]
        ),
        compiler_params=pltpu.CompilerParams(use_tc_tiling_on_sc=False,
                                             needs_layout_passes=False),
    )
    def k(ids_hbm, tp_hbm, out_hbm, *bufs):
        idxs = bufs[:_NBUF]
        rows_v, stage_v = bufs[_NBUF], bufs[_NBUF + 1]
        gsems = bufs[_NBUF + 2:2 * _NBUF + 2]
        isems = bufs[2 * _NBUF + 2:]
        wid = lax.axis_index("s") * nc + lax.axis_index("c")
        base = wid * IPW
        lane_iota = lax.iota(jnp.int32, _LANES)
        zeros16 = jnp.zeros((_LANES,), jnp.int32)

        def idx_copy(it, p):
            return pltpu.make_async_copy(
                ids_hbm.at[pl.ds(it * L, L)], idxs[p].at[pl.ds(0, L)],
                isems[p])

        PRS = (_WB // 8).bit_length() - 1

        def remap_ids(p):
            for q in range(LP // _LANES):
                v = idxs[p][pl.ds(q * _LANES, _LANES)]
                r = v & (_WB - 1)
                rho = (v & ~(_WB - 1)) + ((r & (_WB // 8 - 1)) << 3) + (r >> PRS)
                idxs[p][pl.ds(q * _LANES, _LANES)] = rho

        def gather_parts(p):
            parts = []
            for off in range(0, L, _IDXCAP):
                n = min(_IDXCAP, L - off)
                parts.append((idxs[p].at[pl.ds(off, n)],
                              rows_v.at[p, pl.ds(off, n)]))
            return parts

        def start_gathers(p):
            for idx_s, dst_s in gather_parts(p):
                pltpu.async_copy(tp_hbm.at[idx_s], dst_s, gsems[p])

        def wait_gathers(p):
            for idx_s, dst_s in gather_parts(p):
                pltpu.make_async_copy(tp_hbm.at[idx_s], dst_s, gsems[p]).wait()

        for u in range(_NBUF - 1):
            idx_copy(base + u, u).start()
            idx_copy(base + u, u).wait()
            remap_ids(u)
            start_gathers(u)
        idx_copy(base + _NBUF - 1, _NBUF - 1).start()

        def quad_body(i4, carry):
            for p in range(_NBUF):
                it_off = i4 * _NBUF + p
                it = base + it_off

                @pl.when(it_off + _NBUF - 1 < IPW)
                def _():
                    idx_copy(it + _NBUF - 1, (p + _NBUF - 1) % _NBUF).wait()
                    remap_ids((p + _NBUF - 1) % _NBUF)
                    start_gathers((p + _NBUF - 1) % _NBUF)

                wait_gathers(p)

                @pl.when(it_off + _NBUF < IPW)
                def _():
                    idx_copy(it + _NBUF, p).start()

                def acc_body(r, acc):
                    return acc + rows_v[p, r, pl.ds(0, _LANES)]

                z = jnp.zeros((_LANES,), jnp.float32)
                acc = lax.fori_loop(0, L, acc_body, z, unroll=8)

                g = lax.rem(it_off, GB)
                plsc.store_scatter(stage_v, [lane_iota, zeros16 + g], acc)

                @pl.when(lax.rem(it_off + 1, GB) == 0)
                def _():
                    dst = pl.multiple_of(it + 1 - GB, GB)
                    pltpu.sync_copy(stage_v, out_hbm.at[:, pl.ds(dst, GB)])
            return carry

        lax.fori_loop(0, IPW // _NBUF, quad_body, 0)

    return k(ids_flat, tprime)


def _loss_tc(sums_cm, labels, fc_b, L):
    C, B = sums_cm.shape
    BB = 4096
    nb = B // BB
    inv = float(1.0 / L)

    def body(sum_ref, lab_ref, b_ref, logits_ref, loss_ref):
        i = pl.program_id(0)
        logits = sum_ref[...] * inv + b_ref[...]
        logits_ref[...] = logits
        m = jnp.max(logits, axis=0, keepdims=True)
        lse = jnp.log(jnp.sum(jnp.exp(logits - m), axis=0, keepdims=True)) + m
        onehot = lab_ref[...] == lax.broadcasted_iota(jnp.int32, logits.shape, 0)
        ll = jnp.sum(jnp.where(onehot, logits, 0.0), axis=0, keepdims=True)
        part = jnp.sum(lse - ll)

        @pl.when(i == 0)
        def _():
            loss_ref[0, 0] = 0.0

        loss_ref[0, 0] += part

        @pl.when(i == nb - 1)
        def _():
            loss_ref[0, 0] = loss_ref[0, 0] / B

    logits_cm, loss = pl.pallas_call(
        body,
        grid=(nb,),
        in_specs=[
            pl.BlockSpec((C, BB), lambda i: (0, i)),
            pl.BlockSpec((1, BB), lambda i: (0, i)),
            pl.BlockSpec((C, 1), lambda i: (0, 0)),
        ],
        out_specs=[
            pl.BlockSpec((C, BB), lambda i: (0, i)),
            pl.BlockSpec(memory_space=pltpu.SMEM),
        ],
        out_shape=[
            jax.ShapeDtypeStruct((C, B), jnp.float32),
            jax.ShapeDtypeStruct((1, 1), jnp.float32),
        ],
    )(sums_cm, labels.reshape(1, B), fc_b.reshape(C, 1))
    return loss[0, 0], logits_cm.T


def kernel(input_ids, labels, emb_table, fc_w, fc_b):
    L = input_ids.shape[1]
    tprime = _project_table_tc(emb_table, fc_w)
    sums = _gather_sum_sc(input_ids, tprime)
    loss, logits = _loss_tc(sums, labels, fc_b, L)
    return loss, logits
